# Initial kernel scaffold; baseline (speedup 1.0000x reference)
import jax
import jax.numpy as jnp
from jax.experimental import pallas as pl


def kernel(node_features, edge_features, shared_node_weights, W1n, b1n, Wgn, bgn, W1e, b1e, Wge, bge, edge_index):
    raise NotImplementedError("write your pallas kernel here")


# speedup vs baseline: 1.4424x; 1.4424x over previous
import functools

import jax
import jax.numpy as jnp
from jax import lax
from jax.experimental import pallas as pl
from jax.experimental.pallas import tpu as pltpu
from jax.experimental.pallas import tpu_sc as plsc

N = 10000
E = 320000
DN = 128
DE = 16

NC = 2
NS = 16
NW = NC * NS
EPW = E // NW
K1 = 40
CH1 = EPW // K1
K2 = 80
CH2 = EPW // K2
RPT_A = 632
RPT_B = N - 15 * RPT_A

_HIGH = jax.lax.Precision.HIGHEST


def _sigmoid_den(x):
    return 1.0 + jnp.exp(-x)


def _tabA_body(nf_ref, w_ref, wa_ref, wb_ref, tsrc_ref, tdst_ref):
    x = nf_ref[...]
    tsrc_ref[:, :2 * DN] = jnp.dot(x, wa_ref[...], precision=_HIGH,
                                   preferred_element_type=jnp.float32)
    tsrc_ref[:, 2 * DN:] = w_ref[...]
    tdst_ref[...] = jnp.dot(x, wb_ref[...], precision=_HIGH,
                            preferred_element_type=jnp.float32)


def _node_tables(nf, w, WA, WB):
    BN = 2000
    grid = (N // BN,)
    return pl.pallas_call(
        _tabA_body,
        grid=grid,
        in_specs=[
            pl.BlockSpec((BN, DN), lambda i: (i, 0)),
            pl.BlockSpec((BN, DN), lambda i: (i, 0)),
            pl.BlockSpec((DN, 2 * DN), lambda i: (0, 0)),
            pl.BlockSpec((DN, 2 * DN), lambda i: (0, 0)),
        ],
        out_specs=[
            pl.BlockSpec((BN, 3 * DN), lambda i: (i, 0)),
            pl.BlockSpec((BN, 2 * DN), lambda i: (i, 0)),
        ],
        out_shape=[
            jax.ShapeDtypeStruct((N, 3 * DN), jnp.float32),
            jax.ShapeDtypeStruct((N, 2 * DN), jnp.float32),
        ],
    )(nf, w, WA, WB)


def _tabB_body(ef_ref, wc_ref, bc_ref, wc2_ref, bc2_ref, c_ref, c2_ref):
    ef = ef_ref[...]
    c_ref[...] = jnp.dot(ef, wc_ref[...], precision=_HIGH,
                         preferred_element_type=jnp.float32) + bc_ref[...]
    c2_ref[...] = jnp.dot(ef, wc2_ref[...], precision=_HIGH,
                          preferred_element_type=jnp.float32) + bc2_ref[...]


def _edge_tables(ef, WC, bC, WC2, bC2):
    BE = 8000
    grid = (E // BE,)
    return pl.pallas_call(
        _tabB_body,
        grid=grid,
        in_specs=[
            pl.BlockSpec((BE, DE), lambda i: (i, 0)),
            pl.BlockSpec((DE, 2 * DN), lambda i: (0, 0)),
            pl.BlockSpec((1, 2 * DN), lambda i: (0, 0)),
            pl.BlockSpec((DE, 2 * DE), lambda i: (0, 0)),
            pl.BlockSpec((1, 2 * DE), lambda i: (0, 0)),
        ],
        out_specs=[
            pl.BlockSpec((BE, 2 * DN), lambda i: (i, 0)),
            pl.BlockSpec((BE, 2 * DE), lambda i: (i, 0)),
        ],
        out_shape=[
            jax.ShapeDtypeStruct((E, 2 * DN), jnp.float32),
            jax.ShapeDtypeStruct((E, 2 * DE), jnp.float32),
        ],
    )(ef, WC, bC, WC2, bC2)


def _sc_messages_body(tsrc_hbm, tdst_hbm, c_hbm, src_hbm, dst_hbm, zero_hbm,
                      out_hbm, idx_s, idx_d, gsrc, gdst, cbuf, msg, acc):
    cid = lax.axis_index("c")
    sid = lax.axis_index("s")
    wid = cid * NS + sid

    @pl.when(sid < NS - 1)
    def _():
        pltpu.sync_copy(zero_hbm.at[pl.ds(sid * RPT_A, RPT_A)],
                        acc.at[pl.ds(sid * RPT_A, RPT_A)])

    @pl.when(sid == NS - 1)
    def _():
        pltpu.sync_copy(zero_hbm.at[pl.ds(15 * RPT_A, RPT_B)],
                        acc.at[pl.ds(15 * RPT_A, RPT_B)])

    plsc.subcore_barrier()
    base0 = wid * EPW

    @pl.loop(0, CH1)
    def _chunk(ci):
        base = base0 + ci * K1
        pltpu.sync_copy(src_hbm.at[pl.ds(base, K1)], idx_s)
        pltpu.sync_copy(dst_hbm.at[pl.ds(base, K1)], idx_d)
        pltpu.sync_copy(tsrc_hbm.at[idx_s], gsrc)
        pltpu.sync_copy(tdst_hbm.at[idx_d], gdst)
        pltpu.sync_copy(c_hbm.at[pl.ds(base, K1)], cbuf)

        @pl.loop(0, K1)
        def _edge(e):
            for j in range(DN // 16):
                lo = pl.ds(j * 16, 16)
                hi = pl.ds(DN + j * 16, 16)
                wv = gsrc[e, pl.ds(2 * DN + j * 16, 16)]
                core = gsrc[e, lo] + gdst[e, lo] + cbuf[e, lo]
                gate = gsrc[e, hi] + gdst[e, hi] + cbuf[e, hi]
                den = _sigmoid_den(core) * _sigmoid_den(gate)
                msg[e, lo] = core * wv / den

        pltpu.sync_copy(msg, acc.at[idx_d], add=True)

    plsc.subcore_barrier()

    @pl.when(sid < NS - 1)
    def _():
        pltpu.sync_copy(acc.at[pl.ds(sid * RPT_A, RPT_A)],
                        out_hbm.at[cid, pl.ds(sid * RPT_A, RPT_A)])

    @pl.when(sid == NS - 1)
    def _():
        pltpu.sync_copy(acc.at[pl.ds(15 * RPT_A, RPT_B)],
                        out_hbm.at[cid, pl.ds(15 * RPT_A, RPT_B)])


_sc_messages = functools.partial(
    pl.kernel,
    out_type=jax.ShapeDtypeStruct((NC, N, DN), jnp.float32),
    mesh=plsc.VectorSubcoreMesh(core_axis_name="c", subcore_axis_name="s"),
    scratch_types=[
        pltpu.VMEM((K1,), jnp.int32),
        pltpu.VMEM((K1,), jnp.int32),
        pltpu.VMEM((K1, 3 * DN), jnp.float32),
        pltpu.VMEM((K1, 2 * DN), jnp.float32),
        pltpu.VMEM((K1, 2 * DN), jnp.float32),
        pltpu.VMEM((K1, DN), jnp.float32),
        pltpu.VMEM_SHARED((N, DN), jnp.float32),
    ],
)(_sc_messages_body)


def _tabC_body(nf_ref, w_ref, u_ref, wde_ref, nn_ref, t3_ref):
    u = u_ref[0] + u_ref[1]
    nn = nf_ref[...] + w_ref[...] * u
    nn_ref[...] = nn
    t = jnp.dot(nn, wde_ref[...], precision=_HIGH,
                preferred_element_type=jnp.float32)
    t3_ref[...] = jnp.concatenate([t, jnp.zeros_like(t)], axis=1)


def _combine_tables(nf, w, upd2, WDE):
    BN = 2000
    grid = (N // BN,)
    return pl.pallas_call(
        _tabC_body,
        grid=grid,
        in_specs=[
            pl.BlockSpec((BN, DN), lambda i: (i, 0)),
            pl.BlockSpec((BN, DN), lambda i: (i, 0)),
            pl.BlockSpec((NC, BN, DN), lambda i: (0, i, 0)),
            pl.BlockSpec((DN, 4 * DE), lambda i: (0, 0)),
        ],
        out_specs=[
            pl.BlockSpec((BN, DN), lambda i: (i, 0)),
            pl.BlockSpec((BN, DN), lambda i: (i, 0)),
        ],
        out_shape=[
            jax.ShapeDtypeStruct((N, DN), jnp.float32),
            jax.ShapeDtypeStruct((N, DN), jnp.float32),
        ],
    )(nf, w, upd2, WDE)


def _sc_edges_body(t3_hbm, c2_hbm, ef_hbm, src_hbm, dst_hbm,
                   out_hbm, idx_s, idx_d, g3s, g3d, c2b, efb, outb):
    cid = lax.axis_index("c")
    sid = lax.axis_index("s")
    wid = cid * NS + sid
    base0 = wid * EPW

    @pl.loop(0, CH2)
    def _chunk(ci):
        base = base0 + ci * K2
        pltpu.sync_copy(src_hbm.at[pl.ds(base, K2)], idx_s)
        pltpu.sync_copy(dst_hbm.at[pl.ds(base, K2)], idx_d)
        pltpu.sync_copy(t3_hbm.at[idx_s], g3s)
        pltpu.sync_copy(t3_hbm.at[idx_d], g3d)
        pltpu.sync_copy(c2_hbm.at[pl.ds(base, K2)], c2b)
        pltpu.sync_copy(ef_hbm.at[pl.ds(base, K2)], efb)

        @pl.loop(0, K2)
        def _edge(e):
            lo = pl.ds(0, 16)
            hi = pl.ds(16, 16)
            core = g3s[e, lo] + g3d[e, pl.ds(32, 16)] + c2b[e, lo]
            gate = g3s[e, hi] + g3d[e, pl.ds(48, 16)] + c2b[e, hi]
            den = _sigmoid_den(core) * _sigmoid_den(gate)
            outb[e, lo] = efb[e, lo] + core / den

        pltpu.sync_copy(outb, out_hbm.at[pl.ds(base, K2)])


_sc_edges = functools.partial(
    pl.kernel,
    out_type=jax.ShapeDtypeStruct((E, DE), jnp.float32),
    mesh=plsc.VectorSubcoreMesh(core_axis_name="c", subcore_axis_name="s"),
    scratch_types=[
        pltpu.VMEM((K2,), jnp.int32),
        pltpu.VMEM((K2,), jnp.int32),
        pltpu.VMEM((K2, DN), jnp.float32),
        pltpu.VMEM((K2, DN), jnp.float32),
        pltpu.VMEM((K2, 2 * DE), jnp.float32),
        pltpu.VMEM((K2, DE), jnp.float32),
        pltpu.VMEM((K2, DE), jnp.float32),
    ],
)(_sc_edges_body)


def kernel(node_features, edge_features, shared_node_weights, W1n, b1n, Wgn,
           bgn, W1e, b1e, Wge, bge, edge_index):
    src = edge_index[0]
    dst = edge_index[1]

    WA = jnp.concatenate([W1n[:DN], Wgn[:DN]], axis=1)
    WB = jnp.concatenate([W1n[DN + DE:], Wgn[DN + DE:]], axis=1)
    WC = jnp.concatenate([W1n[DN:DN + DE], Wgn[DN:DN + DE]], axis=1)
    bC = jnp.concatenate([b1n, bgn]).reshape(1, 2 * DN)
    WC2 = jnp.concatenate([W1e[DN:DN + DE], Wge[DN:DN + DE]], axis=1)
    bC2 = jnp.concatenate([b1e, bge]).reshape(1, 2 * DE)
    WDE = jnp.concatenate([W1e[:DN], Wge[:DN], W1e[DN + DE:], Wge[DN + DE:]],
                          axis=1)
    zeros = jnp.zeros((N, DN), jnp.float32)

    tsrc, tdst = _node_tables(node_features, shared_node_weights, WA, WB)
    c_e, c2_e = _edge_tables(edge_features, WC, bC, WC2, bC2)
    upd2 = _sc_messages(tsrc, tdst, c_e, src, dst, zeros)
    new_node, t3 = _combine_tables(node_features, shared_node_weights,
                                   upd2, WDE)
    new_edge = _sc_edges(t3, c2_e, edge_features, src, dst)
    return new_node, new_edge


# speedup vs baseline: 1.8525x; 1.2843x over previous
import functools

import jax
import jax.numpy as jnp
from jax import lax
from jax.experimental import pallas as pl
from jax.experimental.pallas import tpu as pltpu
from jax.experimental.pallas import tpu_sc as plsc

N = 10000
E = 320000
DN = 128
DE = 16

NC = 2
NS = 16
NW = NC * NS
EPW = E // NW
K1 = 16
CH1 = EPW // K1
K2 = 80
CH2 = EPW // K2
RPT_A = 632
RPT_B = N - 15 * RPT_A

_HIGH = jax.lax.Precision.HIGHEST


def _sigmoid_den(x):
    return 1.0 + jnp.exp(-x)


def _tabA_body(nf_ref, wa_ref, wb_ref, tsrc_ref, tdst_ref):
    x = nf_ref[...]
    tsrc_ref[...] = jnp.dot(x, wa_ref[...], precision=_HIGH,
                            preferred_element_type=jnp.float32)
    tdst_ref[...] = jnp.dot(x, wb_ref[...], precision=_HIGH,
                            preferred_element_type=jnp.float32)


def _node_tables(nf, WA, WB):
    BN = 2000
    grid = (N // BN,)
    return pl.pallas_call(
        _tabA_body,
        grid=grid,
        in_specs=[
            pl.BlockSpec((BN, DN), lambda i: (i, 0)),
            pl.BlockSpec((DN, 2 * DN), lambda i: (0, 0)),
            pl.BlockSpec((DN, 2 * DN), lambda i: (0, 0)),
        ],
        out_specs=[
            pl.BlockSpec((BN, 2 * DN), lambda i: (i, 0)),
            pl.BlockSpec((BN, 2 * DN), lambda i: (i, 0)),
        ],
        out_shape=[
            jax.ShapeDtypeStruct((N, 2 * DN), jnp.float32),
            jax.ShapeDtypeStruct((N, 2 * DN), jnp.float32),
        ],
    )(nf, WA, WB)


def _tabB_body(ef_ref, wc_ref, bc_ref, wc2_ref, bc2_ref, c_ref, c2_ref):
    ef = ef_ref[...]
    c_ref[...] = jnp.dot(ef, wc_ref[...], precision=_HIGH,
                         preferred_element_type=jnp.float32) + bc_ref[...]
    c2_ref[...] = jnp.dot(ef, wc2_ref[...], precision=_HIGH,
                          preferred_element_type=jnp.float32) + bc2_ref[...]


def _edge_tables(ef, WC, bC, WC2, bC2):
    BE = 8000
    grid = (E // BE,)
    return pl.pallas_call(
        _tabB_body,
        grid=grid,
        in_specs=[
            pl.BlockSpec((BE, DE), lambda i: (i, 0)),
            pl.BlockSpec((DE, 2 * DN), lambda i: (0, 0)),
            pl.BlockSpec((1, 2 * DN), lambda i: (0, 0)),
            pl.BlockSpec((DE, 2 * DE), lambda i: (0, 0)),
            pl.BlockSpec((1, 2 * DE), lambda i: (0, 0)),
        ],
        out_specs=[
            pl.BlockSpec((BE, 2 * DN), lambda i: (i, 0)),
            pl.BlockSpec((BE, 2 * DE), lambda i: (i, 0)),
        ],
        out_shape=[
            jax.ShapeDtypeStruct((E, 2 * DN), jnp.float32),
            jax.ShapeDtypeStruct((E, 2 * DE), jnp.float32),
        ],
    )(ef, WC, bC, WC2, bC2)


def _sc_messages_body(tsrc_hbm, tdst_hbm, w_hbm, c_hbm, src_hbm, dst_hbm,
                      zero_hbm, out_hbm,
                      cb0, gs0, gd0, wbuf0, idxs0, idxd0, semA0, semB0, semC0,
                      cb1, gs1, gd1, wbuf1, idxs1, idxd1, semA1, semB1, semC1,
                      cb2, gs2, gd2, wbuf2, idxs2, idxd2, semA2, semB2, semC2,
                      acc):
    cid = lax.axis_index("c")
    sid = lax.axis_index("s")
    wid = cid * NS + sid

    @pl.when(sid < NS - 1)
    def _():
        pltpu.sync_copy(zero_hbm.at[pl.ds(sid * RPT_A, RPT_A)],
                        acc.at[pl.ds(sid * RPT_A, RPT_A)])

    @pl.when(sid == NS - 1)
    def _():
        pltpu.sync_copy(zero_hbm.at[pl.ds(15 * RPT_A, RPT_B)],
                        acc.at[pl.ds(15 * RPT_A, RPT_B)])

    plsc.subcore_barrier()
    base0 = wid * EPW

    sets = ((cb0, gs0, gd0, wbuf0, idxs0, idxd0, semA0, semB0, semC0),
            (cb1, gs1, gd1, wbuf1, idxs1, idxd1, semA1, semB1, semC1),
            (cb2, gs2, gd2, wbuf2, idxs2, idxd2, semA2, semB2, semC2))

    def phase1(x, s, drain=True):
        cb, gs, gd, wbuf, idxs, idxd, semA, semB, semC = s
        base = base0 + x * K1

        if drain:
            @pl.when(x >= 3)
            def _():
                pltpu.make_async_copy(wbuf, acc.at[pl.ds(0, K1)],
                                      semC).wait()

        pltpu.async_copy(src_hbm.at[pl.ds(base, K1)], idxs, semA)
        pltpu.async_copy(dst_hbm.at[pl.ds(base, K1)], idxd, semA)
        pltpu.async_copy(c_hbm.at[pl.ds(base, K1)], cb, semB)

    def phase2(x, s):
        cb, gs, gd, wbuf, idxs, idxd, semA, semB, semC = s
        base = base0 + x * K1
        pltpu.make_async_copy(src_hbm.at[pl.ds(base, K1)], idxs, semA).wait()
        pltpu.make_async_copy(dst_hbm.at[pl.ds(base, K1)], idxd, semA).wait()
        pltpu.async_copy(tsrc_hbm.at[idxs], gs, semB)
        pltpu.async_copy(tdst_hbm.at[idxd], gd, semB)
        pltpu.async_copy(w_hbm.at[idxs], wbuf, semB)

    def phase3(x, s):
        cb, gs, gd, wbuf, idxs, idxd, semA, semB, semC = s
        pltpu.make_async_copy(c_hbm.at[pl.ds(0, K1)], cb, semB).wait()
        pltpu.make_async_copy(tsrc_hbm.at[pl.ds(0, K1)], gs, semB).wait()
        pltpu.make_async_copy(tdst_hbm.at[pl.ds(0, K1)], gd, semB).wait()
        pltpu.make_async_copy(w_hbm.at[pl.ds(0, K1)], wbuf, semB).wait()

        @pl.loop(0, K1)
        def _edge(e):
            for j in range(DN // 16):
                lo = pl.ds(j * 16, 16)
                hi = pl.ds(DN + j * 16, 16)
                core = cb[e, lo] + gs[e, lo] + gd[e, lo]
                gate = cb[e, hi] + gs[e, hi] + gd[e, hi]
                den = _sigmoid_den(core) * _sigmoid_den(gate)
                wbuf[e, lo] = core * wbuf[e, lo] / den

        pltpu.async_copy(wbuf, acc.at[idxd], semC, add=True)

    def guarded(fn, x, s):
        @pl.when(x < CH1)
        def _():
            fn(x, s)

    phase1(0, sets[0], drain=False)
    phase1(1, sets[1], drain=False)
    phase2(0, sets[0])

    @pl.loop(0, (CH1 + 2) // 3)
    def _triple(i):
        c = i * 3
        guarded(phase1, c + 2, sets[2])
        guarded(phase2, c + 1, sets[1])
        phase3(c + 0, sets[0])
        guarded(phase1, c + 3, sets[0])
        guarded(phase2, c + 2, sets[2])
        guarded(phase3, c + 1, sets[1])
        guarded(phase1, c + 4, sets[1])
        guarded(phase2, c + 3, sets[0])
        guarded(phase3, c + 2, sets[2])

    for s in sets:
        pltpu.make_async_copy(s[3], acc.at[pl.ds(0, K1)], s[8]).wait()

    plsc.subcore_barrier()

    @pl.when(sid < NS - 1)
    def _():
        pltpu.sync_copy(acc.at[pl.ds(sid * RPT_A, RPT_A)],
                        out_hbm.at[cid, pl.ds(sid * RPT_A, RPT_A)])

    @pl.when(sid == NS - 1)
    def _():
        pltpu.sync_copy(acc.at[pl.ds(15 * RPT_A, RPT_B)],
                        out_hbm.at[cid, pl.ds(15 * RPT_A, RPT_B)])


def _sc1_bufset():
    return [
        pltpu.VMEM((K1, 2 * DN), jnp.float32),
        pltpu.VMEM((K1, 2 * DN), jnp.float32),
        pltpu.VMEM((K1, 2 * DN), jnp.float32),
        pltpu.VMEM((K1, DN), jnp.float32),
        pltpu.VMEM((K1,), jnp.int32),
        pltpu.VMEM((K1,), jnp.int32),
        pltpu.SemaphoreType.DMA,
        pltpu.SemaphoreType.DMA,
        pltpu.SemaphoreType.DMA,
    ]


_sc_messages = functools.partial(
    pl.kernel,
    out_type=jax.ShapeDtypeStruct((NC, N, DN), jnp.float32),
    mesh=plsc.VectorSubcoreMesh(core_axis_name="c", subcore_axis_name="s"),
    scratch_types=[
        *_sc1_bufset(),
        *_sc1_bufset(),
        *_sc1_bufset(),
        pltpu.VMEM_SHARED((N, DN), jnp.float32),
    ],
)(_sc_messages_body)


def _tabC_body(nf_ref, w_ref, u_ref, wde_ref, nn_ref, t3_ref):
    u = u_ref[0] + u_ref[1]
    nn = nf_ref[...] + w_ref[...] * u
    nn_ref[...] = nn
    t = jnp.dot(nn, wde_ref[...], precision=_HIGH,
                preferred_element_type=jnp.float32)
    t3_ref[...] = jnp.concatenate([t, jnp.zeros_like(t)], axis=1)


def _combine_tables(nf, w, upd2, WDE):
    BN = 2000
    grid = (N // BN,)
    return pl.pallas_call(
        _tabC_body,
        grid=grid,
        in_specs=[
            pl.BlockSpec((BN, DN), lambda i: (i, 0)),
            pl.BlockSpec((BN, DN), lambda i: (i, 0)),
            pl.BlockSpec((NC, BN, DN), lambda i: (0, i, 0)),
            pl.BlockSpec((DN, 4 * DE), lambda i: (0, 0)),
        ],
        out_specs=[
            pl.BlockSpec((BN, DN), lambda i: (i, 0)),
            pl.BlockSpec((BN, DN), lambda i: (i, 0)),
        ],
        out_shape=[
            jax.ShapeDtypeStruct((N, DN), jnp.float32),
            jax.ShapeDtypeStruct((N, DN), jnp.float32),
        ],
    )(nf, w, upd2, WDE)


def _sc_edges_body(t3_hbm, c2_hbm, ef_hbm, src_hbm, dst_hbm,
                   out_hbm, idx_s, idx_d, g3s, g3d, c2b, efb, outb):
    cid = lax.axis_index("c")
    sid = lax.axis_index("s")
    wid = cid * NS + sid
    base0 = wid * EPW

    @pl.loop(0, CH2)
    def _chunk(ci):
        base = base0 + ci * K2
        pltpu.sync_copy(src_hbm.at[pl.ds(base, K2)], idx_s)
        pltpu.sync_copy(dst_hbm.at[pl.ds(base, K2)], idx_d)
        pltpu.sync_copy(t3_hbm.at[idx_s], g3s)
        pltpu.sync_copy(t3_hbm.at[idx_d], g3d)
        pltpu.sync_copy(c2_hbm.at[pl.ds(base, K2)], c2b)
        pltpu.sync_copy(ef_hbm.at[pl.ds(base, K2)], efb)

        @pl.loop(0, K2)
        def _edge(e):
            lo = pl.ds(0, 16)
            hi = pl.ds(16, 16)
            core = g3s[e, lo] + g3d[e, pl.ds(32, 16)] + c2b[e, lo]
            gate = g3s[e, hi] + g3d[e, pl.ds(48, 16)] + c2b[e, hi]
            den = _sigmoid_den(core) * _sigmoid_den(gate)
            outb[e, lo] = efb[e, lo] + core / den

        pltpu.sync_copy(outb, out_hbm.at[pl.ds(base, K2)])


_sc_edges = functools.partial(
    pl.kernel,
    out_type=jax.ShapeDtypeStruct((E, DE), jnp.float32),
    mesh=plsc.VectorSubcoreMesh(core_axis_name="c", subcore_axis_name="s"),
    scratch_types=[
        pltpu.VMEM((K2,), jnp.int32),
        pltpu.VMEM((K2,), jnp.int32),
        pltpu.VMEM((K2, DN), jnp.float32),
        pltpu.VMEM((K2, DN), jnp.float32),
        pltpu.VMEM((K2, 2 * DE), jnp.float32),
        pltpu.VMEM((K2, DE), jnp.float32),
        pltpu.VMEM((K2, DE), jnp.float32),
    ],
)(_sc_edges_body)


def kernel(node_features, edge_features, shared_node_weights, W1n, b1n, Wgn,
           bgn, W1e, b1e, Wge, bge, edge_index):
    src = edge_index[0]
    dst = edge_index[1]

    WA = jnp.concatenate([W1n[:DN], Wgn[:DN]], axis=1)
    WB = jnp.concatenate([W1n[DN + DE:], Wgn[DN + DE:]], axis=1)
    WC = jnp.concatenate([W1n[DN:DN + DE], Wgn[DN:DN + DE]], axis=1)
    bC = jnp.concatenate([b1n, bgn]).reshape(1, 2 * DN)
    WC2 = jnp.concatenate([W1e[DN:DN + DE], Wge[DN:DN + DE]], axis=1)
    bC2 = jnp.concatenate([b1e, bge]).reshape(1, 2 * DE)
    WDE = jnp.concatenate([W1e[:DN], Wge[:DN], W1e[DN + DE:], Wge[DN + DE:]],
                          axis=1)
    zeros = jnp.zeros((N, DN), jnp.float32)

    tsrc, tdst = _node_tables(node_features, WA, WB)
    c_e, c2_e = _edge_tables(edge_features, WC, bC, WC2, bC2)
    upd2 = _sc_messages(tsrc, tdst, shared_node_weights, c_e, src, dst, zeros)
    new_node, t3 = _combine_tables(node_features, shared_node_weights,
                                   upd2, WDE)
    new_edge = _sc_edges(t3, c2_e, edge_features, src, dst)
    return new_node, new_edge


# speedup vs baseline: 2.2152x; 1.1958x over previous
import functools

import jax
import jax.numpy as jnp
from jax import lax
from jax.experimental import pallas as pl
from jax.experimental.pallas import tpu as pltpu
from jax.experimental.pallas import tpu_sc as plsc

N = 10000
E = 320000
DN = 128
DE = 16

NC = 2
NS = 16
NW = NC * NS
EPW = E // NW
K1 = 16
CH1 = EPW // K1
K2 = 40
CH2 = EPW // K2
RPT_A = 632
RPT_B = N - 15 * RPT_A

_HIGH = jax.lax.Precision.HIGHEST


def _tabA_body(nf_ref, wa_ref, wb_ref, tsrc_ref, tdst_ref):
    x = nf_ref[...]
    tsrc_ref[...] = jnp.dot(x, wa_ref[...], precision=_HIGH,
                            preferred_element_type=jnp.float32)
    tdst_ref[...] = jnp.dot(x, wb_ref[...], precision=_HIGH,
                            preferred_element_type=jnp.float32)


def _node_tables(nf, WA, WB):
    BN = 2000
    grid = (N // BN,)
    return pl.pallas_call(
        _tabA_body,
        grid=grid,
        in_specs=[
            pl.BlockSpec((BN, DN), lambda i: (i, 0)),
            pl.BlockSpec((DN, 2 * DN), lambda i: (0, 0)),
            pl.BlockSpec((DN, 2 * DN), lambda i: (0, 0)),
        ],
        out_specs=[
            pl.BlockSpec((BN, 2 * DN), lambda i: (i, 0)),
            pl.BlockSpec((BN, 2 * DN), lambda i: (i, 0)),
        ],
        out_shape=[
            jax.ShapeDtypeStruct((N, 2 * DN), jnp.float32),
            jax.ShapeDtypeStruct((N, 2 * DN), jnp.float32),
        ],
    )(nf, WA, WB)


def _tabB_body(ef_ref, wc_ref, bc_ref, wc2_ref, bc2_ref, c_ref, c2_ref):
    ef = ef_ref[...]
    c_ref[...] = jnp.dot(ef, wc_ref[...], precision=_HIGH,
                         preferred_element_type=jnp.float32) + bc_ref[...]
    c2_ref[...] = jnp.dot(ef, wc2_ref[...], precision=_HIGH,
                          preferred_element_type=jnp.float32) + bc2_ref[...]


def _edge_tables(ef, WC, bC, WC2, bC2):
    BE = 8000
    grid = (E // BE,)
    return pl.pallas_call(
        _tabB_body,
        grid=grid,
        in_specs=[
            pl.BlockSpec((BE, DE), lambda i: (i, 0)),
            pl.BlockSpec((DE, 2 * DN), lambda i: (0, 0)),
            pl.BlockSpec((1, 2 * DN), lambda i: (0, 0)),
            pl.BlockSpec((DE, 2 * DE), lambda i: (0, 0)),
            pl.BlockSpec((1, 2 * DE), lambda i: (0, 0)),
        ],
        out_specs=[
            pl.BlockSpec((BE, 2 * DN), lambda i: (i, 0)),
            pl.BlockSpec((BE, 2 * DE), lambda i: (i, 0)),
        ],
        out_shape=[
            jax.ShapeDtypeStruct((E, 2 * DN), jnp.float32),
            jax.ShapeDtypeStruct((E, 2 * DE), jnp.float32),
        ],
    )(ef, WC, bC, WC2, bC2)


def _sc_messages_body(tsrc_hbm, tdst_hbm, w_hbm, c_hbm, src_hbm, dst_hbm,
                      zero_hbm, out_hbm,
                      cb0, gs0, gd0, wbuf0, idxs0, idxd0, semA0, semB0, semC0,
                      cb1, gs1, gd1, wbuf1, idxs1, idxd1, semA1, semB1, semC1,
                      cb2, gs2, gd2, wbuf2, idxs2, idxd2, semA2, semB2, semC2,
                      acc):
    cid = lax.axis_index("c")
    sid = lax.axis_index("s")
    wid = cid * NS + sid

    @pl.when(sid < NS - 1)
    def _():
        pltpu.sync_copy(zero_hbm.at[pl.ds(sid * RPT_A, RPT_A)],
                        acc.at[pl.ds(sid * RPT_A, RPT_A)])

    @pl.when(sid == NS - 1)
    def _():
        pltpu.sync_copy(zero_hbm.at[pl.ds(15 * RPT_A, RPT_B)],
                        acc.at[pl.ds(15 * RPT_A, RPT_B)])

    plsc.subcore_barrier()
    base0 = wid * EPW

    sets = ((cb0, gs0, gd0, wbuf0, idxs0, idxd0, semA0, semB0, semC0),
            (cb1, gs1, gd1, wbuf1, idxs1, idxd1, semA1, semB1, semC1),
            (cb2, gs2, gd2, wbuf2, idxs2, idxd2, semA2, semB2, semC2))

    def phase1(x, s, drain=True):
        cb, gs, gd, wbuf, idxs, idxd, semA, semB, semC = s
        base = base0 + x * K1

        if drain:
            @pl.when(x >= 3)
            def _():
                pltpu.make_async_copy(wbuf, acc.at[pl.ds(0, K1)],
                                      semC).wait()

        pltpu.async_copy(src_hbm.at[pl.ds(base, K1)], idxs, semA)
        pltpu.async_copy(dst_hbm.at[pl.ds(base, K1)], idxd, semA)
        pltpu.async_copy(c_hbm.at[pl.ds(base, K1)], cb, semB)

    def phase2(x, s):
        cb, gs, gd, wbuf, idxs, idxd, semA, semB, semC = s
        base = base0 + x * K1
        pltpu.make_async_copy(src_hbm.at[pl.ds(base, K1)], idxs, semA).wait()
        pltpu.make_async_copy(dst_hbm.at[pl.ds(base, K1)], idxd, semA).wait()
        pltpu.async_copy(tsrc_hbm.at[idxs], gs, semB)
        pltpu.async_copy(tdst_hbm.at[idxd], gd, semB)
        pltpu.async_copy(w_hbm.at[idxs], wbuf, semB)

    def phase3(x, s):
        cb, gs, gd, wbuf, idxs, idxd, semA, semB, semC = s
        pltpu.make_async_copy(c_hbm.at[pl.ds(0, K1)], cb, semB).wait()
        pltpu.make_async_copy(tsrc_hbm.at[pl.ds(0, K1)], gs, semB).wait()
        pltpu.make_async_copy(tdst_hbm.at[pl.ds(0, K1)], gd, semB).wait()
        pltpu.make_async_copy(w_hbm.at[pl.ds(0, K1)], wbuf, semB).wait()

        @pl.loop(0, K1)
        def _edge(e):
            for j in range(DN // 16):
                lo = pl.ds(j * 16, 16)
                hi = pl.ds(DN + j * 16, 16)
                nc = cb[e, lo] + gs[e, lo] + gd[e, lo]
                ng = cb[e, hi] + gs[e, hi] + gd[e, hi]
                den = (1.0 + jnp.exp(nc)) * (1.0 + jnp.exp(ng))
                wbuf[e, lo] = nc * wbuf[e, lo] / den

        pltpu.async_copy(wbuf, acc.at[idxd], semC, add=True)

    def guarded(fn, x, s):
        @pl.when(x < CH1)
        def _():
            fn(x, s)

    phase1(0, sets[0], drain=False)
    phase1(1, sets[1], drain=False)
    phase2(0, sets[0])

    @pl.loop(0, (CH1 + 2) // 3)
    def _triple(i):
        c = i * 3
        guarded(phase1, c + 2, sets[2])
        guarded(phase2, c + 1, sets[1])
        phase3(c + 0, sets[0])
        guarded(phase1, c + 3, sets[0])
        guarded(phase2, c + 2, sets[2])
        guarded(phase3, c + 1, sets[1])
        guarded(phase1, c + 4, sets[1])
        guarded(phase2, c + 3, sets[0])
        guarded(phase3, c + 2, sets[2])

    for s in sets:
        pltpu.make_async_copy(s[3], acc.at[pl.ds(0, K1)], s[8]).wait()

    plsc.subcore_barrier()

    @pl.when(sid < NS - 1)
    def _():
        pltpu.sync_copy(acc.at[pl.ds(sid * RPT_A, RPT_A)],
                        out_hbm.at[cid, pl.ds(sid * RPT_A, RPT_A)])

    @pl.when(sid == NS - 1)
    def _():
        pltpu.sync_copy(acc.at[pl.ds(15 * RPT_A, RPT_B)],
                        out_hbm.at[cid, pl.ds(15 * RPT_A, RPT_B)])


def _sc1_bufset():
    return [
        pltpu.VMEM((K1, 2 * DN), jnp.float32),
        pltpu.VMEM((K1, 2 * DN), jnp.float32),
        pltpu.VMEM((K1, 2 * DN), jnp.float32),
        pltpu.VMEM((K1, DN), jnp.float32),
        pltpu.VMEM((K1,), jnp.int32),
        pltpu.VMEM((K1,), jnp.int32),
        pltpu.SemaphoreType.DMA,
        pltpu.SemaphoreType.DMA,
        pltpu.SemaphoreType.DMA,
    ]


_sc_messages = functools.partial(
    pl.kernel,
    out_type=jax.ShapeDtypeStruct((NC, N, DN), jnp.float32),
    mesh=plsc.VectorSubcoreMesh(core_axis_name="c", subcore_axis_name="s"),
    scratch_types=[
        *_sc1_bufset(),
        *_sc1_bufset(),
        *_sc1_bufset(),
        pltpu.VMEM_SHARED((N, DN), jnp.float32),
    ],
)(_sc_messages_body)


def _tabC_body(nf_ref, w_ref, u_ref, wde_ref, nn_ref, t3_ref):
    u = u_ref[0] + u_ref[1]
    nn = nf_ref[...] - w_ref[...] * u
    nn_ref[...] = nn
    t = jnp.dot(nn, wde_ref[...], precision=_HIGH,
                preferred_element_type=jnp.float32)
    t3_ref[...] = jnp.concatenate([t, jnp.zeros_like(t)], axis=1)


def _combine_tables(nf, w, upd2, WDE):
    BN = 2000
    grid = (N // BN,)
    return pl.pallas_call(
        _tabC_body,
        grid=grid,
        in_specs=[
            pl.BlockSpec((BN, DN), lambda i: (i, 0)),
            pl.BlockSpec((BN, DN), lambda i: (i, 0)),
            pl.BlockSpec((NC, BN, DN), lambda i: (0, i, 0)),
            pl.BlockSpec((DN, 4 * DE), lambda i: (0, 0)),
        ],
        out_specs=[
            pl.BlockSpec((BN, DN), lambda i: (i, 0)),
            pl.BlockSpec((BN, DN), lambda i: (i, 0)),
        ],
        out_shape=[
            jax.ShapeDtypeStruct((N, DN), jnp.float32),
            jax.ShapeDtypeStruct((N, DN), jnp.float32),
        ],
    )(nf, w, upd2, WDE)


def _sc_edges_body(t3_hbm, c2_hbm, ef_hbm, src_hbm, dst_hbm, out_hbm,
                   g3s0, g3d0, c2b0, efb0, outb0, ixs0, ixd0, sA0, sB0, sC0,
                   g3s1, g3d1, c2b1, efb1, outb1, ixs1, ixd1, sA1, sB1, sC1,
                   g3s2, g3d2, c2b2, efb2, outb2, ixs2, ixd2, sA2, sB2, sC2):
    cid = lax.axis_index("c")
    sid = lax.axis_index("s")
    wid = cid * NS + sid
    base0 = wid * EPW

    sets = ((g3s0, g3d0, c2b0, efb0, outb0, ixs0, ixd0, sA0, sB0, sC0),
            (g3s1, g3d1, c2b1, efb1, outb1, ixs1, ixd1, sA1, sB1, sC1),
            (g3s2, g3d2, c2b2, efb2, outb2, ixs2, ixd2, sA2, sB2, sC2))

    def phase1(x, s, drain=True):
        g3s, g3d, c2b, efb, outb, ixs, ixd, sA, sB, sC = s
        base = base0 + x * K2

        if drain:
            @pl.when(x >= 3)
            def _():
                pltpu.make_async_copy(outb, out_hbm.at[pl.ds(0, K2)],
                                      sC).wait()

        pltpu.async_copy(src_hbm.at[pl.ds(base, K2)], ixs, sA)
        pltpu.async_copy(dst_hbm.at[pl.ds(base, K2)], ixd, sA)
        pltpu.async_copy(c2_hbm.at[pl.ds(base, K2)], c2b, sB)
        pltpu.async_copy(ef_hbm.at[pl.ds(base, K2)], efb, sB)

    def phase2(x, s):
        g3s, g3d, c2b, efb, outb, ixs, ixd, sA, sB, sC = s
        base = base0 + x * K2
        pltpu.make_async_copy(src_hbm.at[pl.ds(base, K2)], ixs, sA).wait()
        pltpu.make_async_copy(dst_hbm.at[pl.ds(base, K2)], ixd, sA).wait()
        pltpu.async_copy(t3_hbm.at[ixs], g3s, sB)
        pltpu.async_copy(t3_hbm.at[ixd], g3d, sB)

    def phase3(x, s):
        g3s, g3d, c2b, efb, outb, ixs, ixd, sA, sB, sC = s
        base = base0 + x * K2
        pltpu.make_async_copy(c2_hbm.at[pl.ds(0, K2)], c2b, sB).wait()
        pltpu.make_async_copy(ef_hbm.at[pl.ds(0, K2)], efb, sB).wait()
        pltpu.make_async_copy(t3_hbm.at[pl.ds(0, K2)], g3s, sB).wait()
        pltpu.make_async_copy(t3_hbm.at[pl.ds(0, K2)], g3d, sB).wait()

        @pl.loop(0, K2)
        def _edge(e):
            lo = pl.ds(0, 16)
            hi = pl.ds(16, 16)
            nc = g3s[e, lo] + g3d[e, pl.ds(32, 16)] + c2b[e, lo]
            ng = g3s[e, hi] + g3d[e, pl.ds(48, 16)] + c2b[e, hi]
            den = (1.0 + jnp.exp(nc)) * (1.0 + jnp.exp(ng))
            outb[e, lo] = efb[e, lo] - nc / den

        pltpu.async_copy(outb, out_hbm.at[pl.ds(base, K2)], sC)

    def guarded(fn, x, s):
        @pl.when(x < CH2)
        def _():
            fn(x, s)

    phase1(0, sets[0], drain=False)
    phase1(1, sets[1], drain=False)
    phase2(0, sets[0])

    @pl.loop(0, (CH2 + 2) // 3)
    def _triple(i):
        c = i * 3
        guarded(phase1, c + 2, sets[2])
        guarded(phase2, c + 1, sets[1])
        phase3(c + 0, sets[0])
        guarded(phase1, c + 3, sets[0])
        guarded(phase2, c + 2, sets[2])
        guarded(phase3, c + 1, sets[1])
        guarded(phase1, c + 4, sets[1])
        guarded(phase2, c + 3, sets[0])
        guarded(phase3, c + 2, sets[2])

    for s in sets:
        pltpu.make_async_copy(s[4], out_hbm.at[pl.ds(0, K2)], s[9]).wait()


def _sc2_bufset():
    return [
        pltpu.VMEM((K2, DN), jnp.float32),
        pltpu.VMEM((K2, DN), jnp.float32),
        pltpu.VMEM((K2, 2 * DE), jnp.float32),
        pltpu.VMEM((K2, DE), jnp.float32),
        pltpu.VMEM((K2, DE), jnp.float32),
        pltpu.VMEM((K2,), jnp.int32),
        pltpu.VMEM((K2,), jnp.int32),
        pltpu.SemaphoreType.DMA,
        pltpu.SemaphoreType.DMA,
        pltpu.SemaphoreType.DMA,
    ]


_sc_edges = functools.partial(
    pl.kernel,
    out_type=jax.ShapeDtypeStruct((E, DE), jnp.float32),
    mesh=plsc.VectorSubcoreMesh(core_axis_name="c", subcore_axis_name="s"),
    scratch_types=[
        *_sc2_bufset(),
        *_sc2_bufset(),
        *_sc2_bufset(),
    ],
)(_sc_edges_body)


def kernel(node_features, edge_features, shared_node_weights, W1n, b1n, Wgn,
           bgn, W1e, b1e, Wge, bge, edge_index):
    src = edge_index[0]
    dst = edge_index[1]

    WA = -jnp.concatenate([W1n[:DN], Wgn[:DN]], axis=1)
    WB = -jnp.concatenate([W1n[DN + DE:], Wgn[DN + DE:]], axis=1)
    WC = -jnp.concatenate([W1n[DN:DN + DE], Wgn[DN:DN + DE]], axis=1)
    bC = -jnp.concatenate([b1n, bgn]).reshape(1, 2 * DN)
    WC2 = -jnp.concatenate([W1e[DN:DN + DE], Wge[DN:DN + DE]], axis=1)
    bC2 = -jnp.concatenate([b1e, bge]).reshape(1, 2 * DE)
    WDE = -jnp.concatenate([W1e[:DN], Wge[:DN], W1e[DN + DE:], Wge[DN + DE:]],
                           axis=1)
    zeros = jnp.zeros((N, DN), jnp.float32)

    tsrc, tdst = _node_tables(node_features, WA, WB)
    c_e, c2_e = _edge_tables(edge_features, WC, bC, WC2, bC2)
    upd2 = _sc_messages(tsrc, tdst, shared_node_weights, c_e, src, dst, zeros)
    new_node, t3 = _combine_tables(node_features, shared_node_weights,
                                   upd2, WDE)
    new_edge = _sc_edges(t3, c2_e, edge_features, src, dst)
    return new_node, new_edge
